# Initial kernel scaffold; baseline (speedup 1.0000x reference)
import jax
import jax.numpy as jnp
from jax.experimental import pallas as pl


def kernel(x, edge_index, Ws0, Wn0, b0, Ws1, Wn1, b1, Ws2, Wn2, b2, Ws3, Wn3, b3):
    raise NotImplementedError("write your pallas kernel here")


# speedup vs baseline: 9.8121x; 9.8121x over previous
import functools

import jax
import jax.numpy as jnp
from jax import lax
from jax.experimental import pallas as pl
from jax.experimental.pallas import tpu as pltpu
from jax.experimental.pallas import tpu_sc as plsc

N = 10000
E = 320000
D = 128
H = 64

NC = 2
NS = 16
NW = NC * NS
EPW = E // NW
CH = 125
NCHUNK = EPW // CH
NP = 10240
RPT = NP // NS

BN = 2000
GRID = N // BN


def _sc_body(hn_hbm, src_hbm, dst_hbm, z_hbm,
             parts_hbm,
             srcv, dstv, rows, acc, sem):
    c = lax.axis_index("c")
    s = lax.axis_index("s")
    wid = c * NS + s
    pltpu.sync_copy(z_hbm, acc.at[pl.ds(s * RPT, RPT)])
    pltpu.sync_copy(src_hbm.at[wid], srcv)
    pltpu.sync_copy(dst_hbm.at[wid], dstv)
    plsc.subcore_barrier()

    def chunk(j, carry):
        pltpu.async_copy(hn_hbm.at[srcv.at[j]], rows, sem).wait()
        pltpu.sync_copy(rows, acc.at[dstv.at[j]], add=True)
        return carry

    lax.fori_loop(0, NCHUNK, chunk, 0)
    plsc.subcore_barrier()
    pltpu.sync_copy(acc.at[pl.ds(s * RPT, RPT)],
                    parts_hbm.at[c, pl.ds(s * RPT, RPT)])


def _sc_body_deg(hn_hbm, src_hbm, dst_hbm, z_hbm, z8_hbm, ones_hbm,
                 parts_hbm, degp_hbm,
                 srcv, dstv, rows, onesv, acc, dacc, sem):
    c = lax.axis_index("c")
    s = lax.axis_index("s")
    wid = c * NS + s
    pltpu.sync_copy(z_hbm, acc.at[pl.ds(s * RPT, RPT)])
    pltpu.sync_copy(z8_hbm, dacc.at[pl.ds(s * RPT, RPT)])
    pltpu.sync_copy(src_hbm.at[wid], srcv)
    pltpu.sync_copy(dst_hbm.at[wid], dstv)
    pltpu.sync_copy(ones_hbm, onesv)
    plsc.subcore_barrier()

    def chunk(j, carry):
        pltpu.async_copy(hn_hbm.at[srcv.at[j]], rows, sem).wait()
        pltpu.sync_copy(rows, acc.at[dstv.at[j]], add=True)
        pltpu.sync_copy(onesv, dacc.at[dstv.at[j]], add=True)
        return carry

    lax.fori_loop(0, NCHUNK, chunk, 0)
    plsc.subcore_barrier()
    pltpu.sync_copy(acc.at[pl.ds(s * RPT, RPT)],
                    parts_hbm.at[c, pl.ds(s * RPT, RPT)])
    pltpu.sync_copy(dacc.at[pl.ds(s * RPT, RPT)],
                    degp_hbm.at[c, pl.ds(s * RPT, RPT)])


def _make_sc_seg_sum(dout):
    mesh = plsc.VectorSubcoreMesh(core_axis_name="c", subcore_axis_name="s")
    return pl.kernel(
        _sc_body,
        out_type=jax.ShapeDtypeStruct((NC, NP, dout), jnp.float32),
        mesh=mesh,
        scratch_types=[
            pltpu.VMEM((NCHUNK, CH), jnp.int32),
            pltpu.VMEM((NCHUNK, CH), jnp.int32),
            pltpu.VMEM((CH, dout), jnp.float32),
            pltpu.VMEM_SHARED((NP, dout), jnp.float32),
            pltpu.SemaphoreType.DMA,
        ],
        compiler_params=pltpu.CompilerParams(use_tc_tiling_on_sc=False),
    )


def _make_sc_seg_sum_deg(dout):
    mesh = plsc.VectorSubcoreMesh(core_axis_name="c", subcore_axis_name="s")
    return pl.kernel(
        _sc_body_deg,
        out_type=(jax.ShapeDtypeStruct((NC, NP, dout), jnp.float32),
                  jax.ShapeDtypeStruct((NC, NP, 8), jnp.float32)),
        mesh=mesh,
        scratch_types=[
            pltpu.VMEM((NCHUNK, CH), jnp.int32),
            pltpu.VMEM((NCHUNK, CH), jnp.int32),
            pltpu.VMEM((CH, dout), jnp.float32),
            pltpu.VMEM((CH, 8), jnp.float32),
            pltpu.VMEM_SHARED((NP, dout), jnp.float32),
            pltpu.VMEM_SHARED((NP, 8), jnp.float32),
            pltpu.SemaphoreType.DMA,
        ],
        compiler_params=pltpu.CompilerParams(use_tc_tiling_on_sc=False),
    )


def _mm0_body(x_ref, wst_ref, wnt_ref, b_ref, hs_ref, hn_ref):
    h = x_ref[...]
    hs_ref[...] = (jnp.dot(h, wst_ref[...], preferred_element_type=jnp.float32)
                   + b_ref[...])
    hn_ref[...] = jnp.dot(h, wnt_ref[...], preferred_element_type=jnp.float32)


def _comb_body(hsp_ref, parts_ref, degp_ref, wst_ref, wnt_ref, b_ref,
               hs_ref, hn_ref):
    agg = parts_ref[0] + parts_ref[1]
    deg = degp_ref[0][:, 0:1] + degp_ref[1][:, 0:1]
    inv = 1.0 / jnp.maximum(deg, 1.0)
    h = jnp.maximum(hsp_ref[...] + agg * inv, 0.0)
    hs_ref[...] = (jnp.dot(h, wst_ref[...], preferred_element_type=jnp.float32)
                   + b_ref[...])
    hn_ref[...] = jnp.dot(h, wnt_ref[...], preferred_element_type=jnp.float32)


def _final_body(hsp_ref, parts_ref, degp_ref, out_ref):
    agg = parts_ref[0, :N] + parts_ref[1, :N]
    deg = degp_ref[0, :N, 0:1] + degp_ref[1, :N, 0:1]
    inv = 1.0 / jnp.maximum(deg, 1.0)
    val = hsp_ref[...] + agg * inv
    out_ref[...] = val[:, 0]


def _mm0(x, wst, wnt, b):
    din, dout = wst.shape
    return pl.pallas_call(
        _mm0_body,
        grid=(GRID,),
        in_specs=[
            pl.BlockSpec((BN, din), lambda i: (i, 0)),
            pl.BlockSpec((din, dout), lambda i: (0, 0)),
            pl.BlockSpec((din, dout), lambda i: (0, 0)),
            pl.BlockSpec((1, dout), lambda i: (0, 0)),
        ],
        out_specs=[
            pl.BlockSpec((BN, dout), lambda i: (i, 0)),
            pl.BlockSpec((BN, dout), lambda i: (i, 0)),
        ],
        out_shape=[
            jax.ShapeDtypeStruct((N, dout), jnp.float32),
            jax.ShapeDtypeStruct((N, dout), jnp.float32),
        ],
    )(x, wst, wnt, b)


def _comb(hsp, parts, degp, wst, wnt, b):
    din, dout = wst.shape
    return pl.pallas_call(
        _comb_body,
        grid=(GRID,),
        in_specs=[
            pl.BlockSpec((BN, din), lambda i: (i, 0)),
            pl.BlockSpec((NC, BN, din), lambda i: (0, i, 0)),
            pl.BlockSpec((NC, BN, 8), lambda i: (0, i, 0)),
            pl.BlockSpec((din, dout), lambda i: (0, 0)),
            pl.BlockSpec((din, dout), lambda i: (0, 0)),
            pl.BlockSpec((1, dout), lambda i: (0, 0)),
        ],
        out_specs=[
            pl.BlockSpec((BN, dout), lambda i: (i, 0)),
            pl.BlockSpec((BN, dout), lambda i: (i, 0)),
        ],
        out_shape=[
            jax.ShapeDtypeStruct((N, dout), jnp.float32),
            jax.ShapeDtypeStruct((N, dout), jnp.float32),
        ],
    )(hsp, parts, degp, wst, wnt, b)


def _final(hsp, parts, degp):
    return pl.pallas_call(
        _final_body,
        out_shape=jax.ShapeDtypeStruct((N,), jnp.float32),
    )(hsp, parts, degp)


@jax.jit
def kernel(x, edge_index, Ws0, Wn0, b0, Ws1, Wn1, b1, Ws2, Wn2, b2,
           Ws3, Wn3, b3):
    src = edge_index[0].reshape(NW, NCHUNK, CH)
    dst = edge_index[1].reshape(NW, NCHUNK, CH)

    z64 = jnp.zeros((RPT, H), jnp.float32)
    z8 = jnp.zeros((RPT, 8), jnp.float32)
    ones8 = jnp.ones((CH, 8), jnp.float32)

    wst0, wnt0 = Ws0.T, Wn0.T
    wst1, wnt1 = Ws1.T, Wn1.T
    wst2, wnt2 = Ws2.T, Wn2.T
    wst3 = jnp.pad(Ws3.T, ((0, 0), (0, 7)))
    wnt3 = jnp.pad(Wn3.T, ((0, 0), (0, 7)))
    b3p = jnp.pad(b3.reshape(1, 1), ((0, 0), (0, 7)))

    sc64_deg = _make_sc_seg_sum_deg(H)
    sc64 = _make_sc_seg_sum(H)
    sc8 = _make_sc_seg_sum(8)

    hs0, hn0 = _mm0(x, wst0, wnt0, b0.reshape(1, H))
    parts0, degp = sc64_deg(hn0, src, dst, z64, z8, ones8)
    hs1, hn1 = _comb(hs0, parts0, degp, wst1, wnt1, b1.reshape(1, H))
    parts1 = sc64(hn1, src, dst, z64)
    hs2, hn2 = _comb(hs1, parts1, degp, wst2, wnt2, b2.reshape(1, H))
    parts2 = sc64(hn2, src, dst, z64)
    hs3, hn3 = _comb(hs2, parts2, degp, wst3, wnt3, b3p)
    parts3 = sc8(hn3, src, dst, z8)
    return _final(hs3, parts3, degp)


# speedup vs baseline: 15.4725x; 1.5769x over previous
import functools

import jax
import jax.numpy as jnp
from jax import lax
from jax.experimental import pallas as pl
from jax.experimental.pallas import tpu as pltpu
from jax.experimental.pallas import tpu_sc as plsc

N = 10000
E = 320000
D = 128
H = 64

NC = 2
NS = 16
NW = NC * NS
EPW = E // NW
CH = 125
NCHUNK = EPW // CH
NP = 10240
RPT = NP // NS

BN = 2000
GRID = N // BN


K = 4
NBLK = NCHUNK // K


def _gather(hn_hbm, srcv, rows, gsem, j, k):
    return pltpu.make_async_copy(hn_hbm.at[srcv.at[j]], rows.at[k], gsem[k])


def _scat(rows, acc, dstv, ssem, j, k):
    return pltpu.make_async_copy(rows.at[k], acc.at[dstv.at[j]], ssem[k])


def _sc_body(hn_hbm, src_hbm, dst_hbm, z_hbm,
             parts_hbm,
             srcv, dstv, rows, acc, *sems):
    gsem, ssem = sems[:K], sems[K:2 * K]
    c = lax.axis_index("c")
    s = lax.axis_index("s")
    wid = c * NS + s
    pltpu.sync_copy(src_hbm.at[wid], srcv)
    pltpu.sync_copy(dst_hbm.at[wid], dstv)
    pltpu.sync_copy(z_hbm, acc.at[pl.ds(s * RPT, RPT)])
    for k in range(K):
        _gather(hn_hbm, srcv, rows, gsem, k, k).start()
    plsc.subcore_barrier()

    def block(jb, carry, issue_next):
        base = jb * K
        for k in range(K):
            j = base + k
            _gather(hn_hbm, srcv, rows, gsem, j, k).wait()
            _scat(rows, acc, dstv, ssem, j, k).start(add=True)
        if issue_next:
            for k in range(K):
                j = base + k
                _scat(rows, acc, dstv, ssem, j, k).wait()
                _gather(hn_hbm, srcv, rows, gsem, j + K, k).start()
        return carry

    lax.fori_loop(0, NBLK - 1, lambda jb, cy: block(jb, cy, True), 0)
    block(NBLK - 1, 0, False)
    for k in range(K):
        _scat(rows, acc, dstv, ssem, (NBLK - 1) * K + k, k).wait()
    plsc.subcore_barrier()
    pltpu.sync_copy(acc.at[pl.ds(s * RPT, RPT)],
                    parts_hbm.at[c, pl.ds(s * RPT, RPT)])


def _sc_body_deg(hn_hbm, src_hbm, dst_hbm, z_hbm, z8_hbm, ones_hbm,
                 parts_hbm, degp_hbm,
                 srcv, dstv, rows, onesv, acc, dacc, *sems):
    gsem, ssem, dsem = sems[:K], sems[K:2 * K], sems[2 * K:3 * K]
    c = lax.axis_index("c")
    s = lax.axis_index("s")
    wid = c * NS + s
    pltpu.sync_copy(src_hbm.at[wid], srcv)
    pltpu.sync_copy(dst_hbm.at[wid], dstv)
    pltpu.sync_copy(ones_hbm, onesv)
    pltpu.sync_copy(z_hbm, acc.at[pl.ds(s * RPT, RPT)])
    pltpu.sync_copy(z8_hbm, dacc.at[pl.ds(s * RPT, RPT)])
    for k in range(K):
        _gather(hn_hbm, srcv, rows, gsem, k, k).start()
    plsc.subcore_barrier()

    def dscat(j, k):
        return pltpu.make_async_copy(onesv, dacc.at[dstv.at[j]], dsem[k])

    def block(jb, carry, issue_next):
        base = jb * K
        for k in range(K):
            j = base + k
            _gather(hn_hbm, srcv, rows, gsem, j, k).wait()
            _scat(rows, acc, dstv, ssem, j, k).start(add=True)
            dscat(j, k).start(add=True)
        if issue_next:
            for k in range(K):
                j = base + k
                _scat(rows, acc, dstv, ssem, j, k).wait()
                dscat(j, k).wait()
                _gather(hn_hbm, srcv, rows, gsem, j + K, k).start()
        return carry

    lax.fori_loop(0, NBLK - 1, lambda jb, cy: block(jb, cy, True), 0)
    block(NBLK - 1, 0, False)
    for k in range(K):
        j = (NBLK - 1) * K + k
        _scat(rows, acc, dstv, ssem, j, k).wait()
        dscat(j, k).wait()
    plsc.subcore_barrier()
    pltpu.sync_copy(acc.at[pl.ds(s * RPT, RPT)],
                    parts_hbm.at[c, pl.ds(s * RPT, RPT)])
    pltpu.sync_copy(dacc.at[pl.ds(s * RPT, RPT)],
                    degp_hbm.at[c, pl.ds(s * RPT, RPT)])


def _make_sc_seg_sum(dout):
    mesh = plsc.VectorSubcoreMesh(core_axis_name="c", subcore_axis_name="s")
    return pl.kernel(
        _sc_body,
        out_type=jax.ShapeDtypeStruct((NC, NP, dout), jnp.float32),
        mesh=mesh,
        scratch_types=[
            pltpu.VMEM((NCHUNK, CH), jnp.int32),
            pltpu.VMEM((NCHUNK, CH), jnp.int32),
            pltpu.VMEM((K, CH, dout), jnp.float32),
            pltpu.VMEM_SHARED((NP, dout), jnp.float32),
        ] + [pltpu.SemaphoreType.DMA] * (2 * K),
        compiler_params=pltpu.CompilerParams(use_tc_tiling_on_sc=False),
    )


def _make_sc_seg_sum_deg(dout):
    mesh = plsc.VectorSubcoreMesh(core_axis_name="c", subcore_axis_name="s")
    return pl.kernel(
        _sc_body_deg,
        out_type=(jax.ShapeDtypeStruct((NC, NP, dout), jnp.float32),
                  jax.ShapeDtypeStruct((NC, NP, 8), jnp.float32)),
        mesh=mesh,
        scratch_types=[
            pltpu.VMEM((NCHUNK, CH), jnp.int32),
            pltpu.VMEM((NCHUNK, CH), jnp.int32),
            pltpu.VMEM((K, CH, dout), jnp.float32),
            pltpu.VMEM((CH, 8), jnp.float32),
            pltpu.VMEM_SHARED((NP, dout), jnp.float32),
            pltpu.VMEM_SHARED((NP, 8), jnp.float32),
        ] + [pltpu.SemaphoreType.DMA] * (3 * K),
        compiler_params=pltpu.CompilerParams(use_tc_tiling_on_sc=False),
    )


def _mm0_body(x_ref, wst_ref, wnt_ref, b_ref, hs_ref, hn_ref):
    h = x_ref[...]
    hs_ref[...] = (jnp.dot(h, wst_ref[...], preferred_element_type=jnp.float32)
                   + b_ref[...])
    hn_ref[...] = jnp.dot(h, wnt_ref[...], preferred_element_type=jnp.float32)


def _comb_body(hsp_ref, parts_ref, degp_ref, wst_ref, wnt_ref, b_ref,
               hs_ref, hn_ref):
    agg = parts_ref[0] + parts_ref[1]
    deg = degp_ref[0][:, 0:1] + degp_ref[1][:, 0:1]
    inv = 1.0 / jnp.maximum(deg, 1.0)
    h = jnp.maximum(hsp_ref[...] + agg * inv, 0.0)
    hs_ref[...] = (jnp.dot(h, wst_ref[...], preferred_element_type=jnp.float32)
                   + b_ref[...])
    hn_ref[...] = jnp.dot(h, wnt_ref[...], preferred_element_type=jnp.float32)


def _final_body(hsp_ref, parts_ref, degp_ref, out_ref):
    agg = parts_ref[0, :N] + parts_ref[1, :N]
    deg = degp_ref[0, :N, 0:1] + degp_ref[1, :N, 0:1]
    inv = 1.0 / jnp.maximum(deg, 1.0)
    val = hsp_ref[...] + agg * inv
    out_ref[...] = val[:, 0]


def _mm0(x, wst, wnt, b):
    din, dout = wst.shape
    return pl.pallas_call(
        _mm0_body,
        grid=(GRID,),
        in_specs=[
            pl.BlockSpec((BN, din), lambda i: (i, 0)),
            pl.BlockSpec((din, dout), lambda i: (0, 0)),
            pl.BlockSpec((din, dout), lambda i: (0, 0)),
            pl.BlockSpec((1, dout), lambda i: (0, 0)),
        ],
        out_specs=[
            pl.BlockSpec((BN, dout), lambda i: (i, 0)),
            pl.BlockSpec((BN, dout), lambda i: (i, 0)),
        ],
        out_shape=[
            jax.ShapeDtypeStruct((N, dout), jnp.float32),
            jax.ShapeDtypeStruct((N, dout), jnp.float32),
        ],
    )(x, wst, wnt, b)


def _comb(hsp, parts, degp, wst, wnt, b):
    din, dout = wst.shape
    return pl.pallas_call(
        _comb_body,
        grid=(GRID,),
        in_specs=[
            pl.BlockSpec((BN, din), lambda i: (i, 0)),
            pl.BlockSpec((NC, BN, din), lambda i: (0, i, 0)),
            pl.BlockSpec((NC, BN, 8), lambda i: (0, i, 0)),
            pl.BlockSpec((din, dout), lambda i: (0, 0)),
            pl.BlockSpec((din, dout), lambda i: (0, 0)),
            pl.BlockSpec((1, dout), lambda i: (0, 0)),
        ],
        out_specs=[
            pl.BlockSpec((BN, dout), lambda i: (i, 0)),
            pl.BlockSpec((BN, dout), lambda i: (i, 0)),
        ],
        out_shape=[
            jax.ShapeDtypeStruct((N, dout), jnp.float32),
            jax.ShapeDtypeStruct((N, dout), jnp.float32),
        ],
    )(hsp, parts, degp, wst, wnt, b)


def _final(hsp, parts, degp):
    return pl.pallas_call(
        _final_body,
        out_shape=jax.ShapeDtypeStruct((N,), jnp.float32),
    )(hsp, parts, degp)


@jax.jit
def kernel(x, edge_index, Ws0, Wn0, b0, Ws1, Wn1, b1, Ws2, Wn2, b2,
           Ws3, Wn3, b3):
    src = edge_index[0].reshape(NW, NCHUNK, CH)
    dst = edge_index[1].reshape(NW, NCHUNK, CH)

    z64 = jnp.zeros((RPT, H), jnp.float32)
    z8 = jnp.zeros((RPT, 8), jnp.float32)
    ones8 = jnp.ones((CH, 8), jnp.float32)

    wst0, wnt0 = Ws0.T, Wn0.T
    wst1, wnt1 = Ws1.T, Wn1.T
    wst2, wnt2 = Ws2.T, Wn2.T
    wst3 = jnp.pad(Ws3.T, ((0, 0), (0, 7)))
    wnt3 = jnp.pad(Wn3.T, ((0, 0), (0, 7)))
    b3p = jnp.pad(b3.reshape(1, 1), ((0, 0), (0, 7)))

    sc64_deg = _make_sc_seg_sum_deg(H)
    sc64 = _make_sc_seg_sum(H)
    sc8 = _make_sc_seg_sum(8)

    hs0, hn0 = _mm0(x, wst0, wnt0, b0.reshape(1, H))
    parts0, degp = sc64_deg(hn0, src, dst, z64, z8, ones8)
    hs1, hn1 = _comb(hs0, parts0, degp, wst1, wnt1, b1.reshape(1, H))
    parts1 = sc64(hn1, src, dst, z64)
    hs2, hn2 = _comb(hs1, parts1, degp, wst2, wnt2, b2.reshape(1, H))
    parts2 = sc64(hn2, src, dst, z64)
    hs3, hn3 = _comb(hs2, parts2, degp, wst3, wnt3, b3p)
    parts3 = sc8(hn3, src, dst, z8)
    return _final(hs3, parts3, degp)


# speedup vs baseline: 16.3089x; 1.0541x over previous
import jax
import jax.numpy as jnp
from jax import lax
from jax.experimental import pallas as pl
from jax.experimental.pallas import tpu as pltpu
from jax.experimental.pallas import tpu_sc as plsc

N = 10000
E = 320000
D = 128
H = 64

NC = 2
NS = 16
NW = NC * NS
EPW = E // NW
CH = 125
NCHUNK = EPW // CH
NP = 10240
RPT = NP // NS

K = 8
NBLK = NCHUNK // K

BN = 2048
GRID = NP // BN


def _gather(hn_hbm, srcv, rows, gsem, j, k):
    return pltpu.make_async_copy(hn_hbm.at[srcv.at[j]], rows.at[k], gsem[k])


def _scat(rows, acc, dstv, ssem, j, k):
    return pltpu.make_async_copy(rows.at[k], acc.at[dstv.at[j]], ssem[k])


def _sc_body(hn_hbm, src_hbm, dst_hbm, z_hbm,
             parts_hbm,
             srcv, dstv, rows, acc, *sems):
    gsem, ssem = sems[:K], sems[K:2 * K]
    c = lax.axis_index("c")
    s = lax.axis_index("s")
    wid = c * NS + s
    pltpu.sync_copy(src_hbm.at[wid], srcv)
    pltpu.sync_copy(dst_hbm.at[wid], dstv)
    pltpu.sync_copy(z_hbm, acc.at[pl.ds(s * RPT, RPT)])
    for k in range(K):
        _gather(hn_hbm, srcv, rows, gsem, k, k).start()
    plsc.subcore_barrier()

    def block(jb, carry, issue_next):
        base = jb * K
        for k in range(K):
            j = base + k
            _gather(hn_hbm, srcv, rows, gsem, j, k).wait()
            _scat(rows, acc, dstv, ssem, j, k).start(add=True)
        if issue_next:
            for k in range(K):
                j = base + k
                _scat(rows, acc, dstv, ssem, j, k).wait()
                _gather(hn_hbm, srcv, rows, gsem, j + K, k).start()
        return carry

    lax.fori_loop(0, NBLK - 1, lambda jb, cy: block(jb, cy, True), 0)
    block(NBLK - 1, 0, False)
    for k in range(K):
        _scat(rows, acc, dstv, ssem, (NBLK - 1) * K + k, k).wait()
    plsc.subcore_barrier()
    pltpu.sync_copy(acc.at[pl.ds(s * RPT, RPT)],
                    parts_hbm.at[c, pl.ds(s * RPT, RPT)])


def _sc_body_deg(hn_hbm, src_hbm, dst_hbm, z_hbm, z1_hbm, ones_hbm,
                 parts_hbm, degp_hbm,
                 srcv, dstv, rows, onesv, acc, dacc, *sems):
    gsem, ssem, dsem = sems[:K], sems[K:2 * K], sems[2 * K:3 * K]
    c = lax.axis_index("c")
    s = lax.axis_index("s")
    wid = c * NS + s
    pltpu.sync_copy(src_hbm.at[wid], srcv)
    pltpu.sync_copy(dst_hbm.at[wid], dstv)
    pltpu.sync_copy(ones_hbm, onesv)
    pltpu.sync_copy(z_hbm, acc.at[pl.ds(s * RPT, RPT)])
    pltpu.sync_copy(z1_hbm, dacc.at[pl.ds(s * RPT, RPT)])
    for k in range(K):
        _gather(hn_hbm, srcv, rows, gsem, k, k).start()
    plsc.subcore_barrier()

    def dscat(j, k):
        return pltpu.make_async_copy(onesv, dacc.at[dstv.at[j]], dsem[k])

    def block(jb, carry, issue_next):
        base = jb * K
        for k in range(K):
            j = base + k
            _gather(hn_hbm, srcv, rows, gsem, j, k).wait()
            _scat(rows, acc, dstv, ssem, j, k).start(add=True)
            dscat(j, k).start(add=True)
        if issue_next:
            for k in range(K):
                j = base + k
                _scat(rows, acc, dstv, ssem, j, k).wait()
                dscat(j, k).wait()
                _gather(hn_hbm, srcv, rows, gsem, j + K, k).start()
        return carry

    lax.fori_loop(0, NBLK - 1, lambda jb, cy: block(jb, cy, True), 0)
    block(NBLK - 1, 0, False)
    for k in range(K):
        j = (NBLK - 1) * K + k
        _scat(rows, acc, dstv, ssem, j, k).wait()
        dscat(j, k).wait()
    plsc.subcore_barrier()
    pltpu.sync_copy(acc.at[pl.ds(s * RPT, RPT)],
                    parts_hbm.at[c, pl.ds(s * RPT, RPT)])
    pltpu.sync_copy(dacc.at[pl.ds(s * RPT, RPT)],
                    degp_hbm.at[c, pl.ds(s * RPT, RPT)])


_SC_MESH = dict(core_axis_name="c", subcore_axis_name="s")


def _make_sc_seg_sum(dout):
    rows_t = (pltpu.VMEM((K, CH), jnp.float32) if dout == 0 else
              pltpu.VMEM((K, CH, dout), jnp.float32))
    acc_t = (pltpu.VMEM_SHARED((NP,), jnp.float32) if dout == 0 else
             pltpu.VMEM_SHARED((NP, dout), jnp.float32))
    out_t = (jax.ShapeDtypeStruct((NC, NP), jnp.float32) if dout == 0 else
             jax.ShapeDtypeStruct((NC, NP, dout), jnp.float32))
    return pl.kernel(
        _sc_body,
        out_type=out_t,
        mesh=plsc.VectorSubcoreMesh(**_SC_MESH),
        scratch_types=[
            pltpu.VMEM((NCHUNK, CH), jnp.int32),
            pltpu.VMEM((NCHUNK, CH), jnp.int32),
            rows_t,
            acc_t,
        ] + [pltpu.SemaphoreType.DMA] * (2 * K),
        compiler_params=pltpu.CompilerParams(use_tc_tiling_on_sc=False),
    )


def _make_sc_seg_sum_deg(dout):
    return pl.kernel(
        _sc_body_deg,
        out_type=(jax.ShapeDtypeStruct((NC, NP, dout), jnp.float32),
                  jax.ShapeDtypeStruct((NC, NP), jnp.float32)),
        mesh=plsc.VectorSubcoreMesh(**_SC_MESH),
        scratch_types=[
            pltpu.VMEM((NCHUNK, CH), jnp.int32),
            pltpu.VMEM((NCHUNK, CH), jnp.int32),
            pltpu.VMEM((K, CH, dout), jnp.float32),
            pltpu.VMEM((CH,), jnp.float32),
            pltpu.VMEM_SHARED((NP, dout), jnp.float32),
            pltpu.VMEM_SHARED((NP,), jnp.float32),
        ] + [pltpu.SemaphoreType.DMA] * (3 * K),
        compiler_params=pltpu.CompilerParams(use_tc_tiling_on_sc=False),
    )


def _mm0_body(x_ref, wst_ref, wnt_ref, b_ref, hs_ref, hn_ref):
    h = x_ref[...]
    hs_ref[...] = (jnp.dot(h, wst_ref[...], preferred_element_type=jnp.float32)
                   + b_ref[...])
    hn_ref[...] = jnp.dot(h, wnt_ref[...], preferred_element_type=jnp.float32)


def _inv_deg(degp_ref):
    deg = degp_ref[0] + degp_ref[1]
    return (1.0 / jnp.maximum(deg, 1.0))[:, None]


def _comb_body(hsp_ref, parts_ref, degp_ref, wst_ref, wnt_ref, b_ref,
               hs_ref, hn_ref):
    agg = parts_ref[0] + parts_ref[1]
    h = jnp.maximum(hsp_ref[...] + agg * _inv_deg(degp_ref), 0.0)
    hs_ref[...] = (jnp.dot(h, wst_ref[...], preferred_element_type=jnp.float32)
                   + b_ref[...])
    hn_ref[...] = jnp.dot(h, wnt_ref[...], preferred_element_type=jnp.float32)


def _comb3_body(hsp_ref, parts_ref, degp_ref, wst_ref, wnt_ref, b_ref,
                hs_ref, hn_ref):
    agg = parts_ref[0] + parts_ref[1]
    h = jnp.maximum(hsp_ref[...] + agg * _inv_deg(degp_ref), 0.0)
    hs_ref[...] = (jnp.dot(h, wst_ref[...], preferred_element_type=jnp.float32)
                   + b_ref[...])[:, 0]
    hn_ref[...] = jnp.dot(h, wnt_ref[...],
                          preferred_element_type=jnp.float32)[:, 0]


def _final_body(hsp_ref, parts_ref, degp_ref, out_ref):
    agg = parts_ref[0] + parts_ref[1]
    deg = degp_ref[0] + degp_ref[1]
    out_ref[...] = hsp_ref[...] + agg / jnp.maximum(deg, 1.0)


def _row_spec(din):
    return pl.BlockSpec((BN, din), lambda i: (i, 0))


def _w_spec(din, dout):
    return pl.BlockSpec((din, dout), lambda i: (0, 0))


_P1D = pl.BlockSpec((NC, BN), lambda i: (0, i))


def _mm0(x, wst, wnt, b):
    din, dout = wst.shape
    return pl.pallas_call(
        _mm0_body,
        grid=(GRID,),
        in_specs=[_row_spec(din), _w_spec(din, dout), _w_spec(din, dout),
                  pl.BlockSpec((1, dout), lambda i: (0, 0))],
        out_specs=[_row_spec(dout), _row_spec(dout)],
        out_shape=[jax.ShapeDtypeStruct((N, dout), jnp.float32),
                   jax.ShapeDtypeStruct((N, dout), jnp.float32)],
    )(x, wst, wnt, b)


def _comb(hsp, parts, degp, wst, wnt, b):
    din, dout = wst.shape
    return pl.pallas_call(
        _comb_body,
        grid=(GRID,),
        in_specs=[_row_spec(din),
                  pl.BlockSpec((NC, BN, din), lambda i: (0, i, 0)),
                  _P1D,
                  _w_spec(din, dout), _w_spec(din, dout),
                  pl.BlockSpec((1, dout), lambda i: (0, 0))],
        out_specs=[_row_spec(dout), _row_spec(dout)],
        out_shape=[jax.ShapeDtypeStruct((N, dout), jnp.float32),
                   jax.ShapeDtypeStruct((N, dout), jnp.float32)],
    )(hsp, parts, degp, wst, wnt, b)


def _comb3(hsp, parts, degp, wst, wnt, b):
    din, dout = wst.shape
    v1 = pl.BlockSpec((BN,), lambda i: (i,))
    return pl.pallas_call(
        _comb3_body,
        grid=(GRID,),
        in_specs=[_row_spec(din),
                  pl.BlockSpec((NC, BN, din), lambda i: (0, i, 0)),
                  _P1D,
                  _w_spec(din, dout), _w_spec(din, dout),
                  pl.BlockSpec((1, dout), lambda i: (0, 0))],
        out_specs=[v1, v1],
        out_shape=[jax.ShapeDtypeStruct((N,), jnp.float32),
                   jax.ShapeDtypeStruct((N,), jnp.float32)],
    )(hsp, parts, degp, wst, wnt, b)


def _final(hsp, parts, degp):
    v1 = pl.BlockSpec((BN,), lambda i: (i,))
    return pl.pallas_call(
        _final_body,
        grid=(GRID,),
        in_specs=[v1, _P1D, _P1D],
        out_specs=v1,
        out_shape=jax.ShapeDtypeStruct((N,), jnp.float32),
    )(hsp, parts, degp)


@jax.jit
def kernel(x, edge_index, Ws0, Wn0, b0, Ws1, Wn1, b1, Ws2, Wn2, b2,
           Ws3, Wn3, b3):
    src = edge_index[0].reshape(NW, NCHUNK, CH)
    dst = edge_index[1].reshape(NW, NCHUNK, CH)

    z64 = jnp.zeros((RPT, H), jnp.float32)
    z1 = jnp.zeros((RPT,), jnp.float32)
    ones1 = jnp.ones((CH,), jnp.float32)

    wst3 = jnp.pad(Ws3.T, ((0, 0), (0, 7)))
    wnt3 = jnp.pad(Wn3.T, ((0, 0), (0, 7)))
    b3p = jnp.pad(b3.reshape(1, 1), ((0, 0), (0, 7)))

    sc64_deg = _make_sc_seg_sum_deg(H)
    sc64 = _make_sc_seg_sum(H)
    sc1 = _make_sc_seg_sum(0)

    hs0, hn0 = _mm0(x, Ws0.T, Wn0.T, b0.reshape(1, H))
    parts0, degp = sc64_deg(hn0, src, dst, z64, z1, ones1)
    hs1, hn1 = _comb(hs0, parts0, degp, Ws1.T, Wn1.T, b1.reshape(1, H))
    parts1 = sc64(hn1, src, dst, z64)
    hs2, hn2 = _comb(hs1, parts1, degp, Ws2.T, Wn2.T, b2.reshape(1, H))
    parts2 = sc64(hn2, src, dst, z64)
    hs3, hn3 = _comb3(hs2, parts2, degp, wst3, wnt3, b3p)
    parts3 = sc1(hn3, src, dst, z1)
    return _final(hs3, parts3, degp)
